# Initial kernel scaffold; baseline (speedup 1.0000x reference)
import jax
import jax.numpy as jnp
from jax.experimental import pallas as pl


def kernel(conv0_w, conv0_b, conv1_w, conv1_b, conv2_w, conv2_b, conv3_w, conv3_b, conv4_w, conv4_b, conv5_w, conv5_b, emb1_w, emb1_b, emb2_w, emb2_b, emb3_w, emb3_b, mid_w, mid_b, fin_w, fin_b, x):
    raise NotImplementedError("write your pallas kernel here")


# speedup vs baseline: 1.8262x; 1.8262x over previous
import jax
import jax.numpy as jnp
from jax.experimental import pallas as pl
from jax.experimental.pallas import tpu as pltpu

_BATCH = 32
_FRAMES = 10

_LAYERS = (
    (48, 32, 1, 32, True, 4),
    (24, 16, 32, 64, True, 8),
    (12, 8, 64, 128, False, 16),
    (12, 8, 128, 128, True, 16),
    (6, 4, 128, 256, False, 16),
    (6, 4, 256, 256, True, 16),
)


def _banded(w, cin, cout, wsp):
    w3 = w.reshape(3, 3, cin, cout)
    mats = []
    for dh in range(3):
        m = jnp.zeros((wsp * cin, wsp * cout), w.dtype)
        for t in range(3):
            m = m + jnp.kron(jnp.eye(wsp, k=1 - t, dtype=w.dtype), w3[dh, t])
        mats.append(m)
    return jnp.stack(mats).astype(jnp.bfloat16)


def _shift_rows(a, d):
    if d == -1:
        return jnp.concatenate([a[-1:], a[:-1]], axis=0)
    return jnp.concatenate([a[1:], a[:1]], axis=0)


def _conv_kernel(x_ref, m_ref, b_ref, o_ref, *, hsp):
    f32 = jnp.float32
    bf16 = jnp.bfloat16
    a = x_ref[...]
    acc = jnp.dot(a.astype(bf16), m_ref[1], preferred_element_type=f32)
    h = jax.lax.rem(jax.lax.broadcasted_iota(jnp.int32, a.shape, 0), hsp)
    up = jnp.where(h == 0, 0.0, _shift_rows(a, -1))
    acc = acc + jnp.dot(up.astype(bf16), m_ref[0], preferred_element_type=f32)
    dn = jnp.where(h == hsp - 1, 0.0, _shift_rows(a, 1))
    acc = acc + jnp.dot(dn.astype(bf16), m_ref[2], preferred_element_type=f32)
    o_ref[...] = jnp.maximum(acc + b_ref[...], 0.0)


import functools


def _conv_layer(a2d, m, br, hsp, elems):
    rows, k = a2d.shape
    n = m.shape[2]
    rows_step = rows // (_BATCH // elems)
    return pl.pallas_call(
        functools.partial(_conv_kernel, hsp=hsp),
        out_shape=jax.ShapeDtypeStruct((rows, n), jnp.float32),
        grid=(_BATCH // elems,),
        in_specs=[
            pl.BlockSpec((rows_step, k), lambda i: (i, 0)),
            pl.BlockSpec(m.shape, lambda i: (0, 0, 0)),
            pl.BlockSpec(br.shape, lambda i: (0, 0)),
        ],
        out_specs=pl.BlockSpec((rows_step, n), lambda i: (i, 0)),
        compiler_params=pltpu.CompilerParams(
            dimension_semantics=("parallel",),
            vmem_limit_bytes=58 * 1024 * 1024,
        ),
    )(a2d, m, br)


def _xla_pool(a2d, hsp, win, cout):
    a = a2d.reshape(_BATCH * _FRAMES, hsp // 2, 2, win // 2, 2, cout)
    a = jnp.max(a, axis=(2, 4))
    return a.reshape(_BATCH * _FRAMES * (hsp // 2), (win // 2) * cout)


def _mlp_kernel(f_ref, w1, b1, w2, b2, w3, b3, o_ref):
    f32 = jnp.float32
    bf16 = jnp.bfloat16
    h = jnp.dot(f_ref[...].astype(bf16), w1[...], preferred_element_type=f32) + b1[...]
    h = jnp.maximum(h, 0.0)
    h = jnp.dot(h.astype(bf16), w2[...], preferred_element_type=f32) + b2[...]
    h = jnp.maximum(h, 0.0)
    h = jnp.dot(h.astype(bf16), w3[...], preferred_element_type=f32) + b3[...]
    o_ref[...] = jnp.maximum(h, 0.0)


def _head_kernel(e_ref, wm, bm, wf, bf_, o_ref):
    f32 = jnp.float32
    bf16 = jnp.bfloat16
    h = jnp.dot(e_ref[...].astype(bf16), wm[...], preferred_element_type=f32) + bm[...]
    h = jnp.dot(h.astype(bf16), wf[...], preferred_element_type=f32) + bf_[...]
    o_ref[...] = h


def kernel(conv0_w, conv0_b, conv1_w, conv1_b, conv2_w, conv2_b,
           conv3_w, conv3_b, conv4_w, conv4_b, conv5_w, conv5_b,
           emb1_w, emb1_b, emb2_w, emb2_b, emb3_w, emb3_b,
           mid_w, mid_b, fin_w, fin_b, x):
    f32 = jnp.float32
    bf16 = jnp.bfloat16
    a = x.reshape(_BATCH * _FRAMES * 48, 32)

    conv_ws = (conv0_w, conv1_w, conv2_w, conv3_w, conv4_w, conv5_w)
    conv_bs = (conv0_b, conv1_b, conv2_b, conv3_b, conv4_b, conv5_b)
    for (hsp, win, cin, cout, pool_after, elems), w, b in zip(
            _LAYERS, conv_ws, conv_bs):
        m = _banded(w, cin, cout, win)
        br = jnp.tile(b, win)[None, :]
        a = _conv_layer(a, m, br, hsp, elems)
        if pool_after:
            a = _xla_pool(a, hsp, win, cout)

    feat = a.reshape(_BATCH * _FRAMES, 1536)

    emb = pl.pallas_call(
        _mlp_kernel,
        out_shape=jax.ShapeDtypeStruct((_BATCH * _FRAMES, 128), f32),
        grid=(2,),
        in_specs=[
            pl.BlockSpec((_BATCH * 5, 1536), lambda i: (i, 0)),
            pl.BlockSpec((1536, 1024), lambda i: (0, 0)),
            pl.BlockSpec((1, 1024), lambda i: (0, 0)),
            pl.BlockSpec((1024, 1024), lambda i: (0, 0)),
            pl.BlockSpec((1, 1024), lambda i: (0, 0)),
            pl.BlockSpec((1024, 128), lambda i: (0, 0)),
            pl.BlockSpec((1, 128), lambda i: (0, 0)),
        ],
        out_specs=pl.BlockSpec((_BATCH * 5, 128), lambda i: (i, 0)),
        compiler_params=pltpu.CompilerParams(
            dimension_semantics=("parallel",),
            vmem_limit_bytes=58 * 1024 * 1024,
        ),
    )(feat, emb1_w.astype(bf16), emb1_b[None, :],
      emb2_w.astype(bf16), emb2_b[None, :],
      emb3_w.astype(bf16), emb3_b[None, :])

    head_in = emb.reshape(_BATCH, 1280)
    return pl.pallas_call(
        _head_kernel,
        out_shape=jax.ShapeDtypeStruct((_BATCH, 512), f32),
    )(head_in, mid_w.astype(bf16), mid_b[None, :],
      fin_w.astype(bf16), fin_b[None, :])


# speedup vs baseline: 4.5848x; 2.5106x over previous
import jax
import jax.numpy as jnp
from jax.experimental import pallas as pl
from jax.experimental.pallas import tpu as pltpu

_ELEMS = 4
_ROWS0 = 480

_LAYERS = (
    (48, 32, 1, 32, True, 1920),
    (24, 16, 32, 64, True, 1920),
    (12, 8, 64, 128, False, 1920),
    (12, 8, 128, 128, True, 1920),
    (6, 4, 128, 256, False, 1920),
    (6, 4, 256, 256, True, 1920),
)


def _banded(w, cin, cout, wsp):
    w3 = w.reshape(3, 3, cin, cout)
    mats = []
    for dh in range(3):
        m = jnp.zeros((wsp * cin, wsp * cout), w.dtype)
        for t in range(3):
            m = m + jnp.kron(jnp.eye(wsp, k=1 - t, dtype=w.dtype), w3[dh, t])
        mats.append(m)
    return jnp.stack(mats).astype(jnp.bfloat16)


def _tiled_bias(b, wsp):
    return jnp.tile(b, wsp)[None, :]


def _shift_rows(a, d):
    if d == -1:
        return jnp.concatenate([a[-1:], a[:-1]], axis=0)
    return jnp.concatenate([a[1:], a[:1]], axis=0)


def _conv3x3(a, m_ref, b_ref, hsp):
    f32 = jnp.float32
    bf16 = jnp.bfloat16
    acc = jnp.dot(a.astype(bf16), m_ref[1], preferred_element_type=f32)
    h = jax.lax.rem(jax.lax.broadcasted_iota(jnp.int32, a.shape, 0), hsp)
    up = jnp.where(h == 0, 0.0, _shift_rows(a, -1))
    acc = acc + jnp.dot(up.astype(bf16), m_ref[0], preferred_element_type=f32)
    dn = jnp.where(h == hsp - 1, 0.0, _shift_rows(a, 1))
    acc = acc + jnp.dot(dn.astype(bf16), m_ref[2], preferred_element_type=f32)
    return jnp.maximum(acc + b_ref[...], 0.0)


def _pool2x2(a, win, c):
    rows, lanes = a.shape
    ph = jnp.max(a.reshape(rows // 2, 2, lanes), axis=1)
    parts = [
        jnp.maximum(ph[:, (2 * j) * c:(2 * j + 1) * c],
                    ph[:, (2 * j + 1) * c:(2 * j + 2) * c])
        for j in range(win // 2)
    ]
    return jnp.concatenate(parts, axis=1)


def _trunk_kernel(x_ref, m0, b0, m1, b1, m2, b2, m3, b3, m4, b4, m5, b5, o_ref):
    a = x_ref[...].reshape(_ELEMS * _ROWS0, 32)
    layer_params = ((m0, b0), (m1, b1), (m2, b2), (m3, b3), (m4, b4), (m5, b5))
    for (hsp, win, _cin, cout, pool_after, max_rows), (m_ref, b_ref) in zip(
            _LAYERS, layer_params):
        rows = a.shape[0]
        nchunks = rows // max_rows if rows > max_rows else 1
        csize = rows // nchunks
        outs = []
        for k in range(nchunks):
            y = _conv3x3(a[k * csize:(k + 1) * csize, :], m_ref, b_ref, hsp)
            if pool_after:
                y = _pool2x2(y, win, cout)
            outs.append(y)
        a = outs[0] if len(outs) == 1 else jnp.concatenate(outs, axis=0)
    o_ref[...] = a


def _mlp_kernel(f_ref, w1, b1, w2, b2, w3, b3, o_ref):
    f32 = jnp.float32
    bf16 = jnp.bfloat16
    h = jnp.dot(f_ref[...].astype(bf16), w1[...], preferred_element_type=f32) + b1[...]
    h = jnp.maximum(h, 0.0)
    h = jnp.dot(h.astype(bf16), w2[...], preferred_element_type=f32) + b2[...]
    h = jnp.maximum(h, 0.0)
    h = jnp.dot(h.astype(bf16), w3[...], preferred_element_type=f32) + b3[...]
    o_ref[...] = jnp.maximum(h, 0.0)


def _head_kernel(e_ref, wm, bm, wf, bf_, o_ref):
    f32 = jnp.float32
    bf16 = jnp.bfloat16
    h = jnp.dot(e_ref[...].astype(bf16), wm[...], preferred_element_type=f32) + bm[...]
    h = jnp.dot(h.astype(bf16), wf[...], preferred_element_type=f32) + bf_[...]
    o_ref[...] = h


def kernel(conv0_w, conv0_b, conv1_w, conv1_b, conv2_w, conv2_b,
           conv3_w, conv3_b, conv4_w, conv4_b, conv5_w, conv5_b,
           emb1_w, emb1_b, emb2_w, emb2_b, emb3_w, emb3_b,
           mid_w, mid_b, fin_w, fin_b, x):
    f32 = jnp.float32
    bf16 = jnp.bfloat16
    batch = x.shape[0]
    xr = x.reshape(batch, _ROWS0, 32)

    conv_ws = (conv0_w, conv1_w, conv2_w, conv3_w, conv4_w, conv5_w)
    conv_bs = (conv0_b, conv1_b, conv2_b, conv3_b, conv4_b, conv5_b)
    trunk_args = []
    trunk_specs = []
    for (hsp, win, cin, cout, _pool, _mr), w, b in zip(_LAYERS, conv_ws, conv_bs):
        m = _banded(w, cin, cout, win)
        br = _tiled_bias(b, win)
        trunk_args += [m, br]
        trunk_specs += [
            pl.BlockSpec(m.shape, lambda i: (0, 0, 0)),
            pl.BlockSpec(br.shape, lambda i: (0, 0)),
        ]

    conv_out = pl.pallas_call(
        _trunk_kernel,
        out_shape=jax.ShapeDtypeStruct((batch * 30, 512), f32),
        grid=(batch // _ELEMS,),
        in_specs=[pl.BlockSpec((_ELEMS, _ROWS0, 32), lambda i: (i, 0, 0))] + trunk_specs,
        out_specs=pl.BlockSpec((_ELEMS * 30, 512), lambda i: (i, 0)),
        compiler_params=pltpu.CompilerParams(
            dimension_semantics=("parallel",),
            vmem_limit_bytes=58 * 1024 * 1024,
        ),
    )(xr, *trunk_args)

    feat = conv_out.reshape(batch * 10, 1536)

    emb = pl.pallas_call(
        _mlp_kernel,
        out_shape=jax.ShapeDtypeStruct((batch * 10, 128), f32),
        grid=(2,),
        in_specs=[
            pl.BlockSpec((batch * 5, 1536), lambda i: (i, 0)),
            pl.BlockSpec((1536, 1024), lambda i: (0, 0)),
            pl.BlockSpec((1, 1024), lambda i: (0, 0)),
            pl.BlockSpec((1024, 1024), lambda i: (0, 0)),
            pl.BlockSpec((1, 1024), lambda i: (0, 0)),
            pl.BlockSpec((1024, 128), lambda i: (0, 0)),
            pl.BlockSpec((1, 128), lambda i: (0, 0)),
        ],
        out_specs=pl.BlockSpec((batch * 5, 128), lambda i: (i, 0)),
        compiler_params=pltpu.CompilerParams(
            dimension_semantics=("parallel",),
            vmem_limit_bytes=58 * 1024 * 1024,
        ),
    )(feat, emb1_w.astype(bf16), emb1_b[None, :],
      emb2_w.astype(bf16), emb2_b[None, :],
      emb3_w.astype(bf16), emb3_b[None, :])

    head_in = emb.reshape(batch, 1280)
    return pl.pallas_call(
        _head_kernel,
        out_shape=jax.ShapeDtypeStruct((batch, 512), f32),
    )(head_in, mid_w.astype(bf16), mid_b[None, :],
      fin_w.astype(bf16), fin_b[None, :])
